# Initial kernel scaffold; baseline (speedup 1.0000x reference)
import jax
import jax.numpy as jnp
from jax.experimental import pallas as pl


def kernel(logits, targets):
    raise NotImplementedError("write your pallas kernel here")


# speedup vs baseline: 35.0897x; 35.0897x over previous
import functools

import jax
import jax.numpy as jnp
from jax import lax
from jax.experimental import pallas as pl
from jax.experimental.pallas import tpu as pltpu
from jax.experimental.pallas import tpu_sc as plsc

R = 64
N = 262144
KSEL = 26214
L = 16
CH = 16384
NCH = N // CH
VR = CH // L
J1 = 1024
J2 = 1024
SHIFT1 = 21
SHIFT2 = 11
MASK2 = J2 - 1
ROWS_PER_TILE = 2


def _bce_body(x_ref, t_ref, o_ref):
    x = x_ref[...]
    t = t_ref[...]
    o_ref[...] = jnp.maximum(x, 0.0) - x * t + jnp.log1p(jnp.exp(-jnp.abs(x)))


def _bce(x, t):
    return pl.pallas_call(
        _bce_body,
        grid=(R,),
        in_specs=[
            pl.BlockSpec((1, 2048, 128), lambda i: (i, 0, 0)),
            pl.BlockSpec((1, 2048, 128), lambda i: (i, 0, 0)),
        ],
        out_specs=pl.BlockSpec((1, 2048, 128), lambda i: (i, 0, 0)),
        out_shape=jax.ShapeDtypeStruct((R, 2048, 128), jnp.float32),
    )(x, t)


def _descending_scan(gather_tot, target, extra_gather=None):
    lane = lax.iota(jnp.int32, L)
    nv = J1 // L

    def body(i, carry):
        cum, cums, found, bsel, n_gt, s_gt = carry
        jv = nv - 1 - i
        tot = gather_tot(jv)
        tots = extra_gather(jv) if extra_gather is not None else jnp.zeros(
            (L,), jnp.float32)
        rv = lax.rev(tot, (0,))
        rvs = lax.rev(tots, (0,))
        cs = jnp.cumsum(rv)
        css = jnp.cumsum(rvs)
        after = cum + cs
        hit = after >= target
        nhit = jnp.max(plsc.all_reduce_population_count(hit))
        p = jnp.max(plsc.all_reduce_ffs(hit))
        sel = lane == p
        af_p = jnp.sum(jnp.where(sel, after, 0))
        rv_p = jnp.sum(jnp.where(sel, rv, 0))
        cs_s_p = jnp.sum(jnp.where(sel, css, 0.0))
        rvs_p = jnp.sum(jnp.where(sel, rvs, 0.0))
        take = jnp.logical_and(nhit > 0, found == 0)
        bsel = jnp.where(take, jv * L + (L - 1) - p, bsel)
        n_gt = jnp.where(take, af_p - rv_p, n_gt)
        s_gt = jnp.where(take, cums + cs_s_p - rvs_p, s_gt)
        found = jnp.where(nhit > 0, 1, found)
        cum = cum + jnp.sum(tot)
        cums = cums + jnp.sum(tots)
        return cum, cums, found, bsel, n_gt, s_gt

    init = (jnp.int32(0), jnp.float32(0.0), jnp.int32(0), jnp.int32(0),
            jnp.int32(0), jnp.float32(0.0))
    _, _, _, bsel, n_gt, s_gt = lax.fori_loop(0, nv, body, init)
    return bsel, n_gt, s_gt


def _sc_select(loss_flat):
    mesh = plsc.VectorSubcoreMesh(core_axis_name="c", subcore_axis_name="s")

    @functools.partial(
        pl.kernel,
        mesh=mesh,
        compiler_params=pltpu.CompilerParams(needs_layout_passes=False),
        out_type=jax.ShapeDtypeStruct((32 * L,), jnp.float32),
        scratch_types=[
            pltpu.VMEM((CH,), jnp.float32),
            pltpu.VMEM((CH,), jnp.float32),
            pltpu.VMEM((J1 * L,), jnp.int32),
            pltpu.VMEM((J2 * L,), jnp.int32),
            pltpu.VMEM((J2 * L,), jnp.float32),
            pltpu.VMEM((L,), jnp.float32),
            pltpu.SemaphoreType.DMA,
            pltpu.SemaphoreType.DMA,
        ],
    )
    def sel(loss_hbm, out_hbm, buf0, buf1, hist1, cnt2, sum2, outv, sem0, sem1):
        wid = lax.axis_index("s") * 2 + lax.axis_index("c")
        lane = lax.iota(jnp.int32, L)
        ones = jnp.ones((L,), jnp.int32)
        zc = jnp.zeros((L,), jnp.int32)
        zs = jnp.zeros((L,), jnp.float32)
        bufs = (buf0, buf1)
        sems = (sem0, sem1)

        out_acc = jnp.zeros((L,), jnp.float32)
        for rr in range(ROWS_PER_TILE):
            row = wid * ROWS_PER_TILE + rr
            base = row * N

            def zero_body(j, _):
                hist1[pl.ds(j * L, L)] = zc
                cnt2[pl.ds(j * L, L)] = zc
                sum2[pl.ds(j * L, L)] = zs
                return 0

            lax.fori_loop(0, J1, zero_body, 0)

            def p1_body(i, _, buf):
                v = buf[pl.ds(i * L, L)]
                bits = plsc.bitcast(v, jnp.int32)
                k1 = lax.shift_right_logical(bits, SHIFT1)
                idx = lax.shift_left(k1, 4) + lane
                plsc.addupdate_scatter(hist1, [idx], ones)
                return 0

            desc = pltpu.async_copy(
                loss_hbm.at[pl.ds(base, CH)], bufs[0], sems[0])
            for c in range(NCH):
                if c + 1 < NCH:
                    nxt = pltpu.async_copy(
                        loss_hbm.at[pl.ds(base + (c + 1) * CH, CH)],
                        bufs[(c + 1) % 2], sems[(c + 1) % 2])
                desc.wait()
                buf = bufs[c % 2]
                lax.fori_loop(0, VR,
                              functools.partial(p1_body, buf=buf), 0,
                              unroll=4)
                if c + 1 < NCH:
                    desc = nxt

            def gather1(jv):
                tot = zc
                for l in range(L):
                    tot = tot + plsc.load_gather(
                        hist1, [jv * (L * L) + lane * L + l])
                return tot

            b1, n_gt1, _ = _descending_scan(gather1, KSEL)
            r1 = KSEL - n_gt1

            def p2_body(i, acc, buf):
                v = buf[pl.ds(i * L, L)]
                bits = plsc.bitcast(v, jnp.int32)
                k1 = lax.shift_right_logical(bits, SHIFT1)
                m_gt = k1 > b1
                m_eq = k1 == b1
                acc = acc + jnp.where(m_gt, v, 0.0)
                k2 = jnp.bitwise_and(
                    lax.shift_right_logical(bits, SHIFT2), MASK2)
                idx = lax.shift_left(k2, 4) + lane
                plsc.addupdate_scatter(cnt2, [idx], ones, mask=m_eq)
                plsc.addupdate_scatter(sum2, [idx], v, mask=m_eq)
                return acc

            acc = jnp.zeros((L,), jnp.float32)
            desc = pltpu.async_copy(
                loss_hbm.at[pl.ds(base, CH)], bufs[0], sems[0])
            for c in range(NCH):
                if c + 1 < NCH:
                    nxt = pltpu.async_copy(
                        loss_hbm.at[pl.ds(base + (c + 1) * CH, CH)],
                        bufs[(c + 1) % 2], sems[(c + 1) % 2])
                desc.wait()
                buf = bufs[c % 2]
                acc = lax.fori_loop(0, VR,
                                    functools.partial(p2_body, buf=buf), acc,
                                    unroll=4)
                if c + 1 < NCH:
                    desc = nxt
            s_gt1 = jnp.sum(acc)

            def gather2c(jv):
                tot = zc
                for l in range(L):
                    tot = tot + plsc.load_gather(
                        cnt2, [jv * (L * L) + lane * L + l])
                return tot

            def gather2s(jv):
                tot = zs
                for l in range(L):
                    tot = tot + plsc.load_gather(
                        sum2, [jv * (L * L) + lane * L + l])
                return tot

            j2, n_gt2, s_gt2 = _descending_scan(gather2c, r1, gather2s)
            r2 = r1 - n_gt2

            lob = lax.shift_left(lax.shift_left(b1, 10) + j2, SHIFT2)
            lov = jnp.full((L,), lob, jnp.int32)
            mid_v = (plsc.bitcast(lov, jnp.float32)
                     + plsc.bitcast(lov + (1 << SHIFT2), jnp.float32)) * 0.5
            mid = jnp.max(mid_v)

            row_sum = s_gt1 + s_gt2 + r2.astype(jnp.float32) * mid
            out_acc = jnp.where(lane == rr, row_sum, out_acc)

        outv[...] = out_acc
        pltpu.sync_copy(outv, out_hbm.at[pl.ds(wid * L, L)])

    return sel(loss_flat)


def _finish_body(p_ref, o_ref):
    o_ref[...] = jnp.sum(p_ref[...]) * (1.0 / (R * KSEL)) * jnp.ones((1, 1))


def _finish(parts):
    return pl.pallas_call(
        _finish_body,
        out_shape=jax.ShapeDtypeStruct((1, 1), jnp.float32),
    )(parts.reshape(4, 128))


@jax.jit
def kernel(logits, targets):
    x = logits.reshape(R, 2048, 128)
    t = targets.reshape(R, 2048, 128)
    loss = _bce(x, t)
    parts = _sc_select(loss.reshape(-1))
    return _finish(parts).reshape(())


# speedup vs baseline: 69.3547x; 1.9765x over previous
import functools

import jax
import jax.numpy as jnp
from jax import lax
from jax.experimental import pallas as pl
from jax.experimental.pallas import tpu as pltpu
from jax.experimental.pallas import tpu_sc as plsc

R = 64
N = 262144
KSEL = 26214
L = 16
CH = 16384
NCH = N // CH
VR = CH // L
FB = 32768
QB = FB // L
ROWS_PER_TILE = 2


def _bce_body(x_ref, t_ref, o_ref):
    x = x_ref[...]
    t = t_ref[...]
    o_ref[...] = jnp.maximum(x, 0.0) - x * t + jnp.log1p(jnp.exp(-jnp.abs(x)))


def _bce(x, t):
    return pl.pallas_call(
        _bce_body,
        grid=(R,),
        in_specs=[
            pl.BlockSpec((1, 2048, 128), lambda i: (i, 0, 0)),
            pl.BlockSpec((1, 2048, 128), lambda i: (i, 0, 0)),
        ],
        out_specs=pl.BlockSpec((1, 2048, 128), lambda i: (i, 0, 0)),
        out_shape=jax.ShapeDtypeStruct((R, 2048, 128), jnp.float32),
    )(x, t)


def _pick_boundary(tot, tots, cum, cums, target):
    lane = lax.iota(jnp.int32, L)
    rv = lax.rev(tot, (0,))
    rvs = lax.rev(tots, (0,))
    cs = jnp.cumsum(rv)
    css = jnp.cumsum(rvs)
    after = cum + cs
    hit = after >= target
    nhit = jnp.max(plsc.all_reduce_population_count(hit))
    p = jnp.max(plsc.all_reduce_ffs(hit))
    sel = lane == p
    af_p = jnp.sum(jnp.where(sel, after, 0))
    rv_p = jnp.sum(jnp.where(sel, rv, 0))
    cs_s_p = jnp.sum(jnp.where(sel, css, 0.0))
    rvs_p = jnp.sum(jnp.where(sel, rvs, 0.0))
    return nhit > 0, p, af_p - rv_p, cums + cs_s_p - rvs_p


def _descending_scan(load_c, load_s, nv, target):
    def body(i, carry):
        cum, cums, found, bsel, n_gt, s_gt = carry
        jv = nv - 1 - i
        tot = load_c(jv)
        tots = load_s(jv)
        anyhit, p, n_gt_here, s_gt_here = _pick_boundary(
            tot, tots, cum, cums, target)
        take = jnp.logical_and(anyhit, found == 0)
        bsel = jnp.where(take, jv * L + (L - 1) - p, bsel)
        n_gt = jnp.where(take, n_gt_here, n_gt)
        s_gt = jnp.where(take, s_gt_here, s_gt)
        found = jnp.where(anyhit, 1, found)
        cum = cum + jnp.sum(tot)
        cums = cums + jnp.sum(tots)
        return cum, cums, found, bsel, n_gt, s_gt

    init = (jnp.int32(0), jnp.float32(0.0), jnp.int32(0), jnp.int32(0),
            jnp.int32(0), jnp.float32(0.0))
    _, _, _, bsel, n_gt, s_gt = lax.fori_loop(0, nv, body, init)
    return bsel, n_gt, s_gt


def _sc_select(loss_flat):
    mesh = plsc.VectorSubcoreMesh(core_axis_name="c", subcore_axis_name="s")

    @functools.partial(
        pl.kernel,
        mesh=mesh,
        compiler_params=pltpu.CompilerParams(needs_layout_passes=False),
        out_type=jax.ShapeDtypeStruct((32 * L,), jnp.float32),
        scratch_types=[
            pltpu.VMEM((CH,), jnp.float32),
            pltpu.VMEM((CH,), jnp.float32),
            pltpu.VMEM((FB,), jnp.int32),
            pltpu.VMEM((FB,), jnp.float32),
            pltpu.VMEM((QB,), jnp.int32),
            pltpu.VMEM((QB,), jnp.float32),
            pltpu.VMEM((L,), jnp.float32),
            pltpu.SemaphoreType.DMA,
            pltpu.SemaphoreType.DMA,
        ],
    )
    def sel(loss_hbm, out_hbm, buf0, buf1, cnt_h, sum_h, blkc, blks, outv,
            sem0, sem1):
        wid = lax.axis_index("s") * 2 + lax.axis_index("c")
        lane = lax.iota(jnp.int32, L)
        ones = jnp.ones((L,), jnp.int32)
        zc = jnp.zeros((L,), jnp.int32)
        zs = jnp.zeros((L,), jnp.float32)
        bufs = (buf0, buf1)
        sems = (sem0, sem1)

        out_acc = jnp.zeros((L,), jnp.float32)
        for rr in range(ROWS_PER_TILE):
            row = wid * ROWS_PER_TILE + rr
            base = row * N

            @plsc.parallel_loop(0, FB // L, unroll=8)
            def _(j):
                cnt_h[pl.ds(j * L, L)] = zc
                sum_h[pl.ds(j * L, L)] = zs

            def do_chunk(buf):
                @plsc.parallel_loop(0, VR, unroll=8)
                def _(i):
                    v = buf[pl.ds(i * L, L)]
                    bits = plsc.bitcast(v, jnp.int32)
                    f = lax.shift_right_logical(bits, 16)
                    idx = lax.shift_left(jnp.bitwise_and(f, L - 1), 11) \
                        + lax.shift_right_logical(f, 4)
                    plsc.addupdate_scatter(cnt_h, [idx], ones)
                    plsc.addupdate_scatter(sum_h, [idx], v)

            desc = pltpu.async_copy(
                loss_hbm.at[pl.ds(base, CH)], bufs[0], sems[0])
            for c in range(NCH):
                if c + 1 < NCH:
                    nxt = pltpu.async_copy(
                        loss_hbm.at[pl.ds(base + (c + 1) * CH, CH)],
                        bufs[(c + 1) % 2], sems[(c + 1) % 2])
                desc.wait()
                do_chunk(bufs[c % 2])
                if c + 1 < NCH:
                    desc = nxt

            @plsc.parallel_loop(0, QB // L, unroll=2)
            def _(t):
                tc = zc
                ts = zs
                for s in range(L):
                    tc = tc + cnt_h[pl.ds(s * QB + t * L, L)]
                    ts = ts + sum_h[pl.ds(s * QB + t * L, L)]
                blkc[pl.ds(t * L, L)] = tc
                blks[pl.ds(t * L, L)] = ts

            qb, n_gtb, s_gtb = _descending_scan(
                lambda jv: blkc[pl.ds(jv * L, L)],
                lambda jv: blks[pl.ds(jv * L, L)],
                QB // L, KSEL)
            rb = KSEL - n_gtb

            fine_c = plsc.load_gather(cnt_h, [lane * QB + qb])
            fine_s = plsc.load_gather(sum_h, [lane * QB + qb])
            _, p, n_gt2, s_gt2 = _pick_boundary(
                fine_c, fine_s, jnp.int32(0), jnp.float32(0.0), rb)
            fstar = qb * L + (L - 1) - p
            r2 = rb - n_gt2

            lob = lax.shift_left(fstar, 16)
            lov = jnp.full((L,), lob, jnp.int32)
            mid_v = (plsc.bitcast(lov, jnp.float32)
                     + plsc.bitcast(lov + (1 << 16), jnp.float32)) * 0.5
            mid = jnp.max(mid_v)

            row_sum = s_gtb + s_gt2 + r2.astype(jnp.float32) * mid
            out_acc = jnp.where(lane == rr, row_sum, out_acc)

        outv[...] = out_acc
        pltpu.sync_copy(outv, out_hbm.at[pl.ds(wid * L, L)])

    return sel(loss_flat)


def _finish_body(p_ref, o_ref):
    o_ref[...] = jnp.sum(p_ref[...]) * (1.0 / (R * KSEL)) * jnp.ones((1, 1))


def _finish(parts):
    return pl.pallas_call(
        _finish_body,
        out_shape=jax.ShapeDtypeStruct((1, 1), jnp.float32),
    )(parts.reshape(4, 128))


@jax.jit
def kernel(logits, targets):
    x = logits.reshape(R, 2048, 128)
    t = targets.reshape(R, 2048, 128)
    loss = _bce(x, t)
    parts = _sc_select(loss.reshape(-1))
    return _finish(parts).reshape(())


# speedup vs baseline: 98.5238x; 1.4206x over previous
import functools

import jax
import jax.numpy as jnp
from jax import lax
from jax.experimental import pallas as pl
from jax.experimental.pallas import tpu as pltpu
from jax.experimental.pallas import tpu_sc as plsc

R = 64
N = 262144
KSEL = 26214
L = 16
CR = 32
NCH = 512 // CR
FB = 32768
QB = FB // L
ROWS_PER_TILE = 2


def _bce_body(x_ref, t_ref, o_ref):
    x = x_ref[...]
    t = t_ref[...]
    o_ref[...] = jnp.maximum(x, 0.0) - x * t + jnp.log1p(jnp.exp(-jnp.abs(x)))


def _bce(x, t):
    spec = pl.BlockSpec((1, 1, 512, 512), lambda i: (i // 4, i % 4, 0, 0))
    return pl.pallas_call(
        _bce_body,
        grid=(R,),
        in_specs=[spec, spec],
        out_specs=spec,
        out_shape=jax.ShapeDtypeStruct((16, 4, 512, 512), jnp.float32),
    )(x, t)


def _pick_boundary(tot, tots, cum, cums, target):
    lane = lax.iota(jnp.int32, L)
    rv = lax.rev(tot, (0,))
    rvs = lax.rev(tots, (0,))
    cs = jnp.cumsum(rv)
    css = jnp.cumsum(rvs)
    after = cum + cs
    hit = after >= target
    nhit = jnp.max(plsc.all_reduce_population_count(hit))
    p = jnp.max(plsc.all_reduce_ffs(hit))
    sel = lane == p
    af_p = jnp.sum(jnp.where(sel, after, 0))
    rv_p = jnp.sum(jnp.where(sel, rv, 0))
    cs_s_p = jnp.sum(jnp.where(sel, css, 0.0))
    rvs_p = jnp.sum(jnp.where(sel, rvs, 0.0))
    return nhit > 0, p, af_p - rv_p, cums + cs_s_p - rvs_p


def _descending_scan(load_c, load_s, nv, target):
    def body(i, carry):
        cum, cums, found, bsel, n_gt, s_gt = carry
        jv = nv - 1 - i
        tot = load_c(jv)
        tots = load_s(jv)
        anyhit, p, n_gt_here, s_gt_here = _pick_boundary(
            tot, tots, cum, cums, target)
        take = jnp.logical_and(anyhit, found == 0)
        bsel = jnp.where(take, jv * L + (L - 1) - p, bsel)
        n_gt = jnp.where(take, n_gt_here, n_gt)
        s_gt = jnp.where(take, s_gt_here, s_gt)
        found = jnp.where(anyhit, 1, found)
        cum = cum + jnp.sum(tot)
        cums = cums + jnp.sum(tots)
        return cum, cums, found, bsel, n_gt, s_gt

    init = (jnp.int32(0), jnp.float32(0.0), jnp.int32(0), jnp.int32(0),
            jnp.int32(0), jnp.float32(0.0))
    _, _, _, bsel, n_gt, s_gt = lax.fori_loop(0, nv, body, init)
    return bsel, n_gt, s_gt


def _sc_select(loss_flat):
    mesh = plsc.VectorSubcoreMesh(core_axis_name="c", subcore_axis_name="s")

    @functools.partial(
        pl.kernel,
        mesh=mesh,
        compiler_params=pltpu.CompilerParams(needs_layout_passes=False),
        out_type=jax.ShapeDtypeStruct((32 * L,), jnp.float32),
        scratch_types=[
            pltpu.VMEM((CR, 512), jnp.float32),
            pltpu.VMEM((CR, 512), jnp.float32),
            pltpu.VMEM((FB,), jnp.int32),
            pltpu.VMEM((FB,), jnp.float32),
            pltpu.VMEM((QB,), jnp.int32),
            pltpu.VMEM((QB,), jnp.float32),
            pltpu.VMEM((L,), jnp.float32),
            pltpu.SemaphoreType.DMA,
            pltpu.SemaphoreType.DMA,
        ],
    )
    def sel(loss_hbm, out_hbm, buf0, buf1, cnt_h, sum_h, blkc, blks, outv,
            sem0, sem1):
        wid = lax.axis_index("s") * 2 + lax.axis_index("c")
        lane = lax.iota(jnp.int32, L)
        ones = jnp.ones((L,), jnp.int32)
        zc = jnp.zeros((L,), jnp.int32)
        zs = jnp.zeros((L,), jnp.float32)

        out_acc = jnp.zeros((L,), jnp.float32)
        for rr in range(ROWS_PER_TILE):
            row = wid * ROWS_PER_TILE + rr
            bb = lax.shift_right_logical(row, 2)
            cc = jnp.bitwise_and(row, 3)

            @plsc.parallel_loop(0, FB // L, unroll=8)
            def _(j):
                cnt_h[pl.ds(j * L, L)] = zc
                sum_h[pl.ds(j * L, L)] = zs

            def do_chunk(buf):
                @plsc.parallel_loop(0, CR, unroll=1)
                def _(i):
                    for j in range(512 // L):
                        v = buf[i, pl.ds(j * L, L)]
                        bits = plsc.bitcast(v, jnp.int32)
                        f = lax.shift_right_logical(bits, 16)
                        idx = lax.shift_left(jnp.bitwise_and(f, L - 1), 11) \
                            + lax.shift_right_logical(f, 4)
                        plsc.addupdate_scatter(cnt_h, [idx], ones)
                        plsc.addupdate_scatter(sum_h, [idx], v)

            def chunk_src(c):
                return loss_hbm.at[bb, cc, pl.ds(c * CR, CR)]

            pltpu.async_copy(chunk_src(0), buf0, sem0)

            def pair_body(p, _):
                c0 = 2 * p
                pltpu.async_copy(chunk_src(c0 + 1), buf1, sem1)
                pltpu.make_async_copy(chunk_src(c0), buf0, sem0).wait()
                do_chunk(buf0)

                @pl.when(p < NCH // 2 - 1)
                def _():
                    pltpu.async_copy(chunk_src(c0 + 2), buf0, sem0)

                pltpu.make_async_copy(chunk_src(c0 + 1), buf1, sem1).wait()
                do_chunk(buf1)
                return 0

            lax.fori_loop(0, NCH // 2, pair_body, 0)

            @plsc.parallel_loop(0, QB // L, unroll=2)
            def _(t):
                tc = zc
                ts = zs
                for s in range(L):
                    tc = tc + cnt_h[pl.ds(s * QB + t * L, L)]
                    ts = ts + sum_h[pl.ds(s * QB + t * L, L)]
                blkc[pl.ds(t * L, L)] = tc
                blks[pl.ds(t * L, L)] = ts

            qb, n_gtb, s_gtb = _descending_scan(
                lambda jv: blkc[pl.ds(jv * L, L)],
                lambda jv: blks[pl.ds(jv * L, L)],
                QB // L, KSEL)
            rb = KSEL - n_gtb

            fine_c = plsc.load_gather(cnt_h, [lane * QB + qb])
            fine_s = plsc.load_gather(sum_h, [lane * QB + qb])
            _, p, n_gt2, s_gt2 = _pick_boundary(
                fine_c, fine_s, jnp.int32(0), jnp.float32(0.0), rb)
            fstar = qb * L + (L - 1) - p
            r2 = rb - n_gt2

            lob = lax.shift_left(fstar, 16)
            lov = jnp.full((L,), lob, jnp.int32)
            mid_v = (plsc.bitcast(lov, jnp.float32)
                     + plsc.bitcast(lov + (1 << 16), jnp.float32)) * 0.5
            mid = jnp.max(mid_v)

            row_sum = s_gtb + s_gt2 + r2.astype(jnp.float32) * mid
            out_acc = jnp.where(lane == rr, row_sum, out_acc)

        outv[...] = out_acc
        pltpu.sync_copy(outv, out_hbm.at[pl.ds(wid * L, L)])

    return sel(loss_flat)


def _finish_body(p_ref, o_ref):
    o_ref[...] = jnp.sum(p_ref[...]) * (1.0 / (R * KSEL)) * jnp.ones((1, 1))


def _finish(parts):
    return pl.pallas_call(
        _finish_body,
        out_shape=jax.ShapeDtypeStruct((1, 1), jnp.float32),
    )(parts.reshape(4, 128))


@jax.jit
def kernel(logits, targets):
    loss = _bce(logits, targets)
    parts = _sc_select(loss)
    return _finish(parts).reshape(())


# speedup vs baseline: 137.6132x; 1.3968x over previous
import functools

import jax
import jax.numpy as jnp
from jax import lax
from jax.experimental import pallas as pl
from jax.experimental.pallas import tpu as pltpu
from jax.experimental.pallas import tpu_sc as plsc

R = 64
N = 262144
KSEL = 26214
L = 16
CR = 64
NCH = 512 // CR
FB = 32768
QB = FB // L
ROWS_PER_TILE = 2


def _bce_body(x_ref, t_ref, o_ref):
    x = x_ref[...]
    t = t_ref[...]
    o_ref[...] = jnp.maximum(x, 0.0) - x * t + jnp.log1p(jnp.exp(-jnp.abs(x)))


def _bce(x, t):
    spec = pl.BlockSpec((1, 1, 512, 512), lambda i: (i // 4, i % 4, 0, 0))
    return pl.pallas_call(
        _bce_body,
        grid=(R,),
        in_specs=[spec, spec],
        out_specs=spec,
        out_shape=jax.ShapeDtypeStruct((16, 4, 512, 512), jnp.float32),
    )(x, t)


def _pick_boundary(tot, tots, cum, cums, target):
    lane = lax.iota(jnp.int32, L)
    rv = lax.rev(tot, (0,))
    rvs = lax.rev(tots, (0,))
    cs = jnp.cumsum(rv)
    css = jnp.cumsum(rvs)
    after = cum + cs
    hit = after >= target
    nhit = jnp.max(plsc.all_reduce_population_count(hit))
    p = jnp.max(plsc.all_reduce_ffs(hit))
    sel = lane == p
    af_p = jnp.sum(jnp.where(sel, after, 0))
    rv_p = jnp.sum(jnp.where(sel, rv, 0))
    cs_s_p = jnp.sum(jnp.where(sel, css, 0.0))
    rvs_p = jnp.sum(jnp.where(sel, rvs, 0.0))
    return nhit > 0, p, af_p - rv_p, cums + cs_s_p - rvs_p


def _descending_scan(load_c, load_s, nv, target):
    def body(i, carry):
        cum, cums, found, bsel, n_gt, s_gt = carry
        jv = nv - 1 - i
        tot = load_c(jv)
        tots = load_s(jv)
        anyhit, p, n_gt_here, s_gt_here = _pick_boundary(
            tot, tots, cum, cums, target)
        take = jnp.logical_and(anyhit, found == 0)
        bsel = jnp.where(take, jv * L + (L - 1) - p, bsel)
        n_gt = jnp.where(take, n_gt_here, n_gt)
        s_gt = jnp.where(take, s_gt_here, s_gt)
        found = jnp.where(anyhit, 1, found)
        cum = cum + jnp.sum(tot)
        cums = cums + jnp.sum(tots)
        return cum, cums, found, bsel, n_gt, s_gt

    init = (jnp.int32(0), jnp.float32(0.0), jnp.int32(0), jnp.int32(0),
            jnp.int32(0), jnp.float32(0.0))
    _, _, _, bsel, n_gt, s_gt = lax.fori_loop(0, nv, body, init)
    return bsel, n_gt, s_gt


def _sc_select(loss_flat):
    mesh = plsc.VectorSubcoreMesh(core_axis_name="c", subcore_axis_name="s")

    @functools.partial(
        pl.kernel,
        mesh=mesh,
        compiler_params=pltpu.CompilerParams(needs_layout_passes=False),
        out_type=jax.ShapeDtypeStruct((32 * L,), jnp.float32),
        scratch_types=[
            pltpu.VMEM((CR, 512), jnp.float32),
            pltpu.VMEM((CR, 512), jnp.float32),
            pltpu.VMEM((FB,), jnp.int32),
            pltpu.VMEM((QB,), jnp.int32),
            pltpu.VMEM((QB,), jnp.float32),
            pltpu.VMEM((L,), jnp.float32),
            pltpu.SemaphoreType.DMA,
            pltpu.SemaphoreType.DMA,
        ],
    )
    def sel(loss_hbm, out_hbm, buf0, buf1, cnt_h, blkc, blks, outv,
            sem0, sem1):
        wid = lax.axis_index("s") * 2 + lax.axis_index("c")
        lane = lax.iota(jnp.int32, L)
        ones = jnp.ones((L,), jnp.int32)
        zc = jnp.zeros((L,), jnp.int32)
        zs = jnp.zeros((L,), jnp.float32)

        out_acc = jnp.zeros((L,), jnp.float32)
        for rr in range(ROWS_PER_TILE):
            row = wid * ROWS_PER_TILE + rr
            bb = lax.shift_right_logical(row, 2)
            cc = jnp.bitwise_and(row, 3)

            @plsc.parallel_loop(0, FB // L, unroll=8)
            def _(j):
                cnt_h[pl.ds(j * L, L)] = zc

            def do_chunk(buf):
                @plsc.parallel_loop(0, CR, unroll=1)
                def _(i):
                    for j in range(512 // L):
                        v = buf[i, pl.ds(j * L, L)]
                        bits = plsc.bitcast(v, jnp.int32)
                        idx = lax.shift_right_logical(
                            jnp.bitwise_and(bits, 0xF0000), 5) \
                            | lax.shift_right_logical(bits, 20)
                        plsc.addupdate_scatter(cnt_h, [idx], ones)

            def chunk_src(c):
                return loss_hbm.at[bb, cc, pl.ds(c * CR, CR)]

            pltpu.async_copy(chunk_src(0), buf0, sem0)

            def pair_body(p, _):
                c0 = 2 * p
                pltpu.async_copy(chunk_src(c0 + 1), buf1, sem1)
                pltpu.make_async_copy(chunk_src(c0), buf0, sem0).wait()
                do_chunk(buf0)

                @pl.when(p < NCH // 2 - 1)
                def _():
                    pltpu.async_copy(chunk_src(c0 + 2), buf0, sem0)

                pltpu.make_async_copy(chunk_src(c0 + 1), buf1, sem1).wait()
                do_chunk(buf1)
                return 0

            lax.fori_loop(0, NCH // 2, pair_body, 0)

            @plsc.parallel_loop(0, QB // L, unroll=2)
            def _(t):
                tc = zc
                ts = zs
                base_f = lax.shift_left(t, 8) + lane * L
                for s in range(L):
                    c = cnt_h[pl.ds(s * QB + t * L, L)]
                    fcl = jnp.minimum(base_f + s, 0x7F7F)
                    mid = plsc.bitcast(
                        lax.shift_left(fcl, 16) | 0x8000, jnp.float32)
                    tc = tc + c
                    ts = ts + c.astype(jnp.float32) * mid
                blkc[pl.ds(t * L, L)] = tc
                blks[pl.ds(t * L, L)] = ts

            qb, n_gtb, s_gtb = _descending_scan(
                lambda jv: blkc[pl.ds(jv * L, L)],
                lambda jv: blks[pl.ds(jv * L, L)],
                QB // L, KSEL)
            rb = KSEL - n_gtb

            fine_c = plsc.load_gather(cnt_h, [lane * QB + qb])
            fcl = jnp.minimum(qb * L + lane, 0x7F7F)
            mid_f = plsc.bitcast(
                lax.shift_left(fcl, 16) | 0x8000, jnp.float32)
            fine_s = fine_c.astype(jnp.float32) * mid_f
            _, p, n_gt2, s_gt2 = _pick_boundary(
                fine_c, fine_s, jnp.int32(0), jnp.float32(0.0), rb)
            r2 = rb - n_gt2
            mid_star = jnp.sum(jnp.where(lane == (L - 1) - p, mid_f, 0.0))

            row_sum = s_gtb + s_gt2 + r2.astype(jnp.float32) * mid_star
            out_acc = jnp.where(lane == rr, row_sum, out_acc)

        outv[...] = out_acc
        pltpu.sync_copy(outv, out_hbm.at[pl.ds(wid * L, L)])

    return sel(loss_flat)


def _finish_body(p_ref, o_ref):
    o_ref[...] = jnp.sum(p_ref[...]) * (1.0 / (R * KSEL)) * jnp.ones((1, 1))


def _finish(parts):
    return pl.pallas_call(
        _finish_body,
        out_shape=jax.ShapeDtypeStruct((1, 1), jnp.float32),
    )(parts.reshape(4, 128))


@jax.jit
def kernel(logits, targets):
    loss = _bce(logits, targets)
    parts = _sc_select(loss)
    return _finish(parts).reshape(())


# speedup vs baseline: 137.8786x; 1.0019x over previous
import functools

import jax
import jax.numpy as jnp
from jax import lax
from jax.experimental import pallas as pl
from jax.experimental.pallas import tpu as pltpu
from jax.experimental.pallas import tpu_sc as plsc

R = 64
N = 262144
KSEL = 26214
L = 16
CR = 64
NCH = 512 // CR
FB = 32768
QB = FB // L
ROWS_PER_TILE = 2


def _bce_body(x_ref, t_ref, o_ref):
    x = x_ref[...]
    t = t_ref[...]
    o_ref[...] = jnp.maximum(x, 0.0) - x * t + jnp.log1p(jnp.exp(-jnp.abs(x)))


def _bce(x, t):
    spec = pl.BlockSpec((1, 1, 512, 512), lambda i: (i // 4, i % 4, 0, 0))
    return pl.pallas_call(
        _bce_body,
        grid=(R,),
        in_specs=[spec, spec],
        out_specs=spec,
        out_shape=jax.ShapeDtypeStruct((16, 4, 512, 512), jnp.float32),
    )(x, t)


def _pick_boundary(tot, tots, cum, cums, target):
    lane = lax.iota(jnp.int32, L)
    rv = lax.rev(tot, (0,))
    rvs = lax.rev(tots, (0,))
    cs = jnp.cumsum(rv)
    css = jnp.cumsum(rvs)
    after = cum + cs
    hit = after >= target
    nhit = jnp.max(plsc.all_reduce_population_count(hit))
    p = jnp.max(plsc.all_reduce_ffs(hit))
    sel = lane == p
    af_p = jnp.sum(jnp.where(sel, after, 0))
    rv_p = jnp.sum(jnp.where(sel, rv, 0))
    cs_s_p = jnp.sum(jnp.where(sel, css, 0.0))
    rvs_p = jnp.sum(jnp.where(sel, rvs, 0.0))
    return nhit > 0, p, af_p - rv_p, cums + cs_s_p - rvs_p


def _descending_scan(load_c, load_s, nv, target):
    def body(i, carry):
        cum, cums, found, bsel, n_gt, s_gt = carry
        jv = nv - 1 - i
        tot = load_c(jv)
        tots = load_s(jv)
        anyhit, p, n_gt_here, s_gt_here = _pick_boundary(
            tot, tots, cum, cums, target)
        take = jnp.logical_and(anyhit, found == 0)
        bsel = jnp.where(take, jv * L + (L - 1) - p, bsel)
        n_gt = jnp.where(take, n_gt_here, n_gt)
        s_gt = jnp.where(take, s_gt_here, s_gt)
        found = jnp.where(anyhit, 1, found)
        cum = cum + jnp.sum(tot)
        cums = cums + jnp.sum(tots)
        return cum, cums, found, bsel, n_gt, s_gt

    init = (jnp.int32(0), jnp.float32(0.0), jnp.int32(0), jnp.int32(0),
            jnp.int32(0), jnp.float32(0.0))
    _, _, _, bsel, n_gt, s_gt = lax.fori_loop(0, nv, body, init)
    return bsel, n_gt, s_gt


def _sc_select(loss_flat):
    mesh = plsc.VectorSubcoreMesh(core_axis_name="c", subcore_axis_name="s")

    @functools.partial(
        pl.kernel,
        mesh=mesh,
        compiler_params=pltpu.CompilerParams(needs_layout_passes=False),
        out_type=jax.ShapeDtypeStruct((32 * L,), jnp.float32),
        scratch_types=[
            pltpu.VMEM((CR, 512), jnp.float32),
            pltpu.VMEM((CR, 512), jnp.float32),
            pltpu.VMEM((FB,), jnp.int32),
            pltpu.VMEM((QB,), jnp.int32),
            pltpu.VMEM((QB,), jnp.float32),
            pltpu.VMEM((L,), jnp.float32),
            pltpu.SemaphoreType.DMA,
            pltpu.SemaphoreType.DMA,
        ],
    )
    def sel(loss_hbm, out_hbm, buf0, buf1, cnt_h, blkc, blks, outv,
            sem0, sem1):
        wid = lax.axis_index("s") * 2 + lax.axis_index("c")
        lane = lax.iota(jnp.int32, L)
        ones = jnp.ones((L,), jnp.int32)
        zc = jnp.zeros((L,), jnp.int32)
        zs = jnp.zeros((L,), jnp.float32)

        out_acc = jnp.zeros((L,), jnp.float32)
        for rr in range(ROWS_PER_TILE):
            row = wid * ROWS_PER_TILE + rr
            bb = lax.shift_right_logical(row, 2)
            cc = jnp.bitwise_and(row, 3)

            @plsc.parallel_loop(0, FB // L, unroll=8)
            def _(j):
                cnt_h[pl.ds(j * L, L)] = zc

            def do_chunk(buf):
                @plsc.parallel_loop(0, CR, unroll=2)
                def _(i):
                    for j in range(512 // L):
                        v = buf[i, pl.ds(j * L, L)]
                        bits = plsc.bitcast(v, jnp.int32)
                        idx = lax.shift_right_logical(
                            jnp.bitwise_and(bits, 0xF0000), 5) \
                            | lax.shift_right_logical(bits, 20)
                        plsc.addupdate_scatter(cnt_h, [idx], ones)

            def chunk_src(c):
                return loss_hbm.at[bb, cc, pl.ds(c * CR, CR)]

            pltpu.async_copy(chunk_src(0), buf0, sem0)

            def pair_body(p, _):
                c0 = 2 * p
                pltpu.async_copy(chunk_src(c0 + 1), buf1, sem1)
                pltpu.make_async_copy(chunk_src(c0), buf0, sem0).wait()
                do_chunk(buf0)

                @pl.when(p < NCH // 2 - 1)
                def _():
                    pltpu.async_copy(chunk_src(c0 + 2), buf0, sem0)

                pltpu.make_async_copy(chunk_src(c0 + 1), buf1, sem1).wait()
                do_chunk(buf1)
                return 0

            lax.fori_loop(0, NCH // 2, pair_body, 0)

            @plsc.parallel_loop(0, QB // L, unroll=2)
            def _(t):
                tc = zc
                ts = zs
                base_f = lax.shift_left(t, 8) + lane * L
                for s in range(L):
                    c = cnt_h[pl.ds(s * QB + t * L, L)]
                    fcl = jnp.minimum(base_f + s, 0x7F7F)
                    mid = plsc.bitcast(
                        lax.shift_left(fcl, 16) | 0x8000, jnp.float32)
                    tc = tc + c
                    ts = ts + c.astype(jnp.float32) * mid
                blkc[pl.ds(t * L, L)] = tc
                blks[pl.ds(t * L, L)] = ts

            qb, n_gtb, s_gtb = _descending_scan(
                lambda jv: blkc[pl.ds(jv * L, L)],
                lambda jv: blks[pl.ds(jv * L, L)],
                QB // L, KSEL)
            rb = KSEL - n_gtb

            fine_c = plsc.load_gather(cnt_h, [lane * QB + qb])
            fcl = jnp.minimum(qb * L + lane, 0x7F7F)
            mid_f = plsc.bitcast(
                lax.shift_left(fcl, 16) | 0x8000, jnp.float32)
            fine_s = fine_c.astype(jnp.float32) * mid_f
            _, p, n_gt2, s_gt2 = _pick_boundary(
                fine_c, fine_s, jnp.int32(0), jnp.float32(0.0), rb)
            r2 = rb - n_gt2
            mid_star = jnp.sum(jnp.where(lane == (L - 1) - p, mid_f, 0.0))

            row_sum = s_gtb + s_gt2 + r2.astype(jnp.float32) * mid_star
            out_acc = jnp.where(lane == rr, row_sum, out_acc)

        outv[...] = out_acc
        pltpu.sync_copy(outv, out_hbm.at[pl.ds(wid * L, L)])

    return sel(loss_flat)


def _finish_body(p_ref, o_ref):
    o_ref[...] = jnp.sum(p_ref[...]) * (1.0 / (R * KSEL)) * jnp.ones((1, 1))


def _finish(parts):
    return pl.pallas_call(
        _finish_body,
        out_shape=jax.ShapeDtypeStruct((1, 1), jnp.float32),
    )(parts.reshape(4, 128))


@jax.jit
def kernel(logits, targets):
    loss = _bce(logits, targets)
    parts = _sc_select(loss)
    return _finish(parts).reshape(())


# speedup vs baseline: 168.2633x; 1.2204x over previous
import functools

import jax
import jax.numpy as jnp
from jax import lax
from jax.experimental import pallas as pl
from jax.experimental.pallas import tpu as pltpu
from jax.experimental.pallas import tpu_sc as plsc

R = 64
N = 262144
KSEL = 26214
L = 16
CR = 64
NCH = 512 // CR
FB = 32768
QB = FB // L


def _bce_body(x_ref, t_ref, o_ref):
    x = x_ref[...]
    t = t_ref[...]
    o_ref[...] = jnp.maximum(x, 0.0) - x * t + jnp.log1p(jnp.exp(-jnp.abs(x)))


def _bce(x, t, b0, nb):
    spec_in = pl.BlockSpec((1, 1, 512, 512),
                           lambda i: (b0 + i // 4, i % 4, 0, 0))
    spec_out = pl.BlockSpec((1, 1, 512, 512), lambda i: (i // 4, i % 4, 0, 0))
    return pl.pallas_call(
        _bce_body,
        grid=(nb * 4,),
        in_specs=[spec_in, spec_in],
        out_specs=spec_out,
        out_shape=jax.ShapeDtypeStruct((nb, 4, 512, 512), jnp.float32),
    )(x, t)


def _pick_boundary(tot, tots, cum, cums, target):
    lane = lax.iota(jnp.int32, L)
    rv = lax.rev(tot, (0,))
    rvs = lax.rev(tots, (0,))
    cs = jnp.cumsum(rv)
    css = jnp.cumsum(rvs)
    after = cum + cs
    hit = after >= target
    nhit = jnp.max(plsc.all_reduce_population_count(hit))
    p = jnp.max(plsc.all_reduce_ffs(hit))
    sel = lane == p
    af_p = jnp.sum(jnp.where(sel, after, 0))
    rv_p = jnp.sum(jnp.where(sel, rv, 0))
    cs_s_p = jnp.sum(jnp.where(sel, css, 0.0))
    rvs_p = jnp.sum(jnp.where(sel, rvs, 0.0))
    return nhit > 0, p, af_p - rv_p, cums + cs_s_p - rvs_p


def _descending_scan(load_c, load_s, nv, target):
    def body(i, carry):
        cum, cums, found, bsel, n_gt, s_gt = carry
        jv = nv - 1 - i
        tot = load_c(jv)
        tots = load_s(jv)
        anyhit, p, n_gt_here, s_gt_here = _pick_boundary(
            tot, tots, cum, cums, target)
        take = jnp.logical_and(anyhit, found == 0)
        bsel = jnp.where(take, jv * L + (L - 1) - p, bsel)
        n_gt = jnp.where(take, n_gt_here, n_gt)
        s_gt = jnp.where(take, s_gt_here, s_gt)
        found = jnp.where(anyhit, 1, found)
        cum = cum + jnp.sum(tot)
        cums = cums + jnp.sum(tots)
        return cum, cums, found, bsel, n_gt, s_gt

    init = (jnp.int32(0), jnp.float32(0.0), jnp.int32(0), jnp.int32(0),
            jnp.int32(0), jnp.float32(0.0))
    _, _, _, bsel, n_gt, s_gt = lax.fori_loop(0, nv, body, init)
    return bsel, n_gt, s_gt


def _sc_select(loss_flat):
    rows_per_tile = loss_flat.shape[0] * loss_flat.shape[1] // 32
    mesh = plsc.VectorSubcoreMesh(core_axis_name="c", subcore_axis_name="s")

    @functools.partial(
        pl.kernel,
        mesh=mesh,
        compiler_params=pltpu.CompilerParams(needs_layout_passes=False),
        out_type=jax.ShapeDtypeStruct((32 * L,), jnp.float32),
        scratch_types=[
            pltpu.VMEM((CR, 512), jnp.float32),
            pltpu.VMEM((CR, 512), jnp.float32),
            pltpu.VMEM((FB,), jnp.int32),
            pltpu.VMEM((QB,), jnp.int32),
            pltpu.VMEM((QB,), jnp.float32),
            pltpu.VMEM((L,), jnp.float32),
            pltpu.SemaphoreType.DMA,
            pltpu.SemaphoreType.DMA,
        ],
    )
    def sel(loss_hbm, out_hbm, buf0, buf1, cnt_h, blkc, blks, outv,
            sem0, sem1):
        wid = lax.axis_index("s") * 2 + lax.axis_index("c")
        lane = lax.iota(jnp.int32, L)
        ones = jnp.ones((L,), jnp.int32)
        zc = jnp.zeros((L,), jnp.int32)
        zs = jnp.zeros((L,), jnp.float32)

        out_acc = jnp.zeros((L,), jnp.float32)
        for rr in range(rows_per_tile):
            row = wid * rows_per_tile + rr
            bb = lax.shift_right_logical(row, 2)
            cc = jnp.bitwise_and(row, 3)

            @plsc.parallel_loop(0, FB // L, unroll=8)
            def _(j):
                cnt_h[pl.ds(j * L, L)] = zc

            def do_chunk(buf):
                @plsc.parallel_loop(0, CR, unroll=2)
                def _(i):
                    for j in range(512 // L):
                        v = buf[i, pl.ds(j * L, L)]
                        bits = plsc.bitcast(v, jnp.int32)
                        idx = lax.shift_right_logical(
                            jnp.bitwise_and(bits, 0xF0000), 5) \
                            | lax.shift_right_logical(bits, 20)
                        plsc.addupdate_scatter(cnt_h, [idx], ones)

            def chunk_src(c):
                return loss_hbm.at[bb, cc, pl.ds(c * CR, CR)]

            pltpu.async_copy(chunk_src(0), buf0, sem0)

            def pair_body(p, _):
                c0 = 2 * p
                pltpu.async_copy(chunk_src(c0 + 1), buf1, sem1)
                pltpu.make_async_copy(chunk_src(c0), buf0, sem0).wait()
                do_chunk(buf0)

                @pl.when(p < NCH // 2 - 1)
                def _():
                    pltpu.async_copy(chunk_src(c0 + 2), buf0, sem0)

                pltpu.make_async_copy(chunk_src(c0 + 1), buf1, sem1).wait()
                do_chunk(buf1)
                return 0

            lax.fori_loop(0, NCH // 2, pair_body, 0)

            @plsc.parallel_loop(0, QB // L, unroll=2)
            def _(t):
                tc = zc
                ts = zs
                base_f = lax.shift_left(t, 8) + lane * L
                for s in range(L):
                    c = cnt_h[pl.ds(s * QB + t * L, L)]
                    fcl = jnp.minimum(base_f + s, 0x7F7F)
                    mid = plsc.bitcast(
                        lax.shift_left(fcl, 16) | 0x8000, jnp.float32)
                    tc = tc + c
                    ts = ts + c.astype(jnp.float32) * mid
                blkc[pl.ds(t * L, L)] = tc
                blks[pl.ds(t * L, L)] = ts

            qb, n_gtb, s_gtb = _descending_scan(
                lambda jv: blkc[pl.ds(jv * L, L)],
                lambda jv: blks[pl.ds(jv * L, L)],
                QB // L, KSEL)
            rb = KSEL - n_gtb

            fine_c = plsc.load_gather(cnt_h, [lane * QB + qb])
            fcl = jnp.minimum(qb * L + lane, 0x7F7F)
            mid_f = plsc.bitcast(
                lax.shift_left(fcl, 16) | 0x8000, jnp.float32)
            fine_s = fine_c.astype(jnp.float32) * mid_f
            _, p, n_gt2, s_gt2 = _pick_boundary(
                fine_c, fine_s, jnp.int32(0), jnp.float32(0.0), rb)
            r2 = rb - n_gt2
            mid_star = jnp.sum(jnp.where(lane == (L - 1) - p, mid_f, 0.0))

            row_sum = s_gtb + s_gt2 + r2.astype(jnp.float32) * mid_star
            out_acc = jnp.where(lane == rr, row_sum, out_acc)

        outv[...] = out_acc
        pltpu.sync_copy(outv, out_hbm.at[pl.ds(wid * L, L)])

    return sel(loss_flat)


def _finish_body(p0_ref, p1_ref, o_ref):
    o_ref[...] = (jnp.sum(p0_ref[...]) + jnp.sum(p1_ref[...])) \
        * (1.0 / (R * KSEL)) * jnp.ones((1, 1))


def _finish(parts0, parts1):
    return pl.pallas_call(
        _finish_body,
        out_shape=jax.ShapeDtypeStruct((1, 1), jnp.float32),
    )(parts0.reshape(4, 128), parts1.reshape(4, 128))


@jax.jit
def kernel(logits, targets):
    loss0 = _bce(logits, targets, 0, 8)
    loss1 = _bce(logits, targets, 8, 8)
    parts0 = _sc_select(loss0)
    parts1 = _sc_select(loss1)
    return _finish(parts0, parts1).reshape(())
